# Initial kernel scaffold; baseline (speedup 1.0000x reference)
import jax
import jax.numpy as jnp
from jax.experimental import pallas as pl


def kernel(input_ids, token_type_ids, word_table, pos_table, type_table, ln_gamma, ln_beta):
    raise NotImplementedError("write your pallas kernel here")


# speedup vs baseline: 6.7806x; 6.7806x over previous
import functools

import jax
import jax.numpy as jnp
from jax.experimental import pallas as pl
from jax.experimental.pallas import tpu as pltpu
from jax.experimental.pallas import tpu_sc as plsc

_EPS = 1e-12
_GATHER_WINDOW = 128


def _sc_gather(table, ids_2d, n_rows, hidden):
    mesh = plsc.VectorSubcoreMesh(core_axis_name="c", subcore_axis_name="s")
    w = _GATHER_WINDOW

    @functools.partial(
        pl.kernel,
        out_type=jax.ShapeDtypeStruct((n_rows, hidden), jnp.float32),
        mesh=mesh,
    )
    def gather_kernel(table_hbm, idx_hbm, out_hbm):
        def body(i_vmem, o_vmem):
            pltpu.sync_copy(table_hbm.at[i_vmem.at[0]], o_vmem)

        pltpu.emit_pipeline(
            body,
            grid=(n_rows // w,),
            in_specs=[pl.BlockSpec((1, w), lambda i: (0, i))],
            out_specs=[pl.BlockSpec((w, hidden), lambda i: (i, 0))],
            core_axis_name=("c", "s"),
            dimension_semantics=(pltpu.PARALLEL,),
        )(idx_hbm, out_hbm)

    return gather_kernel(table, ids_2d)


def _ln_body(tt_ref, w_ref, pos_ref, type_ref, g_ref, b_ref, o_ref):
    w = w_ref[...]
    ttf = tt_ref[...]
    pos = pos_ref[...]
    t0 = type_ref[0][None, None, :]
    dt = (type_ref[1] - type_ref[0])[None, None, :]
    emb = w + pos[None] + t0 + ttf * dt
    mean = jnp.mean(emb, axis=-1, keepdims=True)
    c = emb - mean
    var = jnp.mean(c * c, axis=-1, keepdims=True)
    y = c * jax.lax.rsqrt(var + _EPS)
    o_ref[...] = y * g_ref[0][None, None, :] + b_ref[0][None, None, :]


def _tc_layernorm(gathered, token_type_f, pos_table, type_table, g2d, b2d):
    bsz, seq = token_type_f.shape[:2]
    hidden = gathered.shape[-1]
    bblk = 16
    grid = (bsz // bblk,)
    return pl.pallas_call(
        _ln_body,
        grid=grid,
        in_specs=[
            pl.BlockSpec((bblk, seq, 1), lambda i: (i, 0, 0)),
            pl.BlockSpec((bblk, seq, hidden), lambda i: (i, 0, 0)),
            pl.BlockSpec((seq, hidden), lambda i: (0, 0)),
            pl.BlockSpec(type_table.shape, lambda i: (0, 0)),
            pl.BlockSpec((1, hidden), lambda i: (0, 0)),
            pl.BlockSpec((1, hidden), lambda i: (0, 0)),
        ],
        out_specs=pl.BlockSpec((bblk, seq, hidden), lambda i: (i, 0, 0)),
        out_shape=jax.ShapeDtypeStruct((bsz, seq, hidden), jnp.float32),
    )(token_type_f, gathered.reshape(bsz, seq, hidden), pos_table,
      type_table, g2d, b2d)


def kernel(input_ids, token_type_ids, word_table, pos_table, type_table,
           ln_gamma, ln_beta):
    bsz, seq = input_ids.shape
    hidden = word_table.shape[1]
    n_rows = bsz * seq
    ids_2d = input_ids.reshape(1, n_rows).astype(jnp.int32)
    gathered = _sc_gather(word_table, ids_2d, n_rows, hidden)
    return _tc_layernorm(
        gathered,
        token_type_ids.astype(jnp.float32).reshape(bsz, seq, 1),
        pos_table,
        type_table,
        ln_gamma.reshape(1, hidden),
        ln_beta.reshape(1, hidden),
    )
